# baseline (speedup 1.0000x reference)
import functools

import jax
import jax.numpy as jnp
from jax import lax
from jax.experimental import pallas as pl
from jax.experimental.pallas import tpu as pltpu
from jax.experimental.pallas import tpu_sc as plsc

_mesh = plsc.VectorSubcoreMesh(core_axis_name="c", subcore_axis_name="s")

B = 32
V = 1_000_000
LANES = 128
BV = 65536
G = BV // LANES
NBLK = (V + BV - 1) // BV
NEG_HUGE = -3.4028235e38
IMAX = 2**31 - 1


def _tc_body(a_ref, x_ref, lp_ref, mode_ref, m_s, s_s, i_s, g_s):
    k = pl.program_id(0)

    @pl.when(k == 0)
    def _():
        m_s[...] = jnp.full((B, LANES), NEG_HUGE, jnp.float32)
        s_s[...] = jnp.zeros((B, LANES), jnp.float32)
        i_s[...] = jnp.zeros((B, LANES), jnp.int32)
        g_s[...] = jnp.zeros((B, LANES), jnp.float32)

    x3 = x_ref[...].reshape(B, G, LANES)
    offs = (k * BV
            + lax.broadcasted_iota(jnp.int32, (B, G, LANES), 1) * LANES
            + lax.broadcasted_iota(jnp.int32, (B, G, LANES), 2))
    x3 = jnp.where(offs < V, x3, NEG_HUGE)

    bm = jnp.max(x3, axis=1)
    m_old = m_s[...]
    m_new = jnp.maximum(m_old, bm)
    bs = jnp.sum(jnp.exp(x3 - m_new[:, None, :]), axis=1)
    s_s[...] = s_s[...] * jnp.exp(m_old - m_new) + bs
    m_s[...] = m_new

    a3 = a_ref[...][:, :, None]
    g_s[...] += jnp.sum(jnp.where(offs == a3, x3, 0.0), axis=1)

    giota = lax.broadcasted_iota(jnp.int32, (B, G, LANES), 1)
    bg = jnp.min(jnp.where(x3 == bm[:, None, :], giota, IMAX), axis=1)
    lane = lax.broadcasted_iota(jnp.int32, (B, LANES), 1)
    elem = k * BV + bg * LANES + lane
    i_s[...] = jnp.where(bm > m_old, elem, i_s[...])

    @pl.when(k == NBLK - 1)
    def _():
        m_l = m_s[...]
        M = jnp.max(m_l, axis=1, keepdims=True)
        S = jnp.sum(s_s[...] * jnp.exp(m_l - M), axis=1, keepdims=True)
        g = jnp.sum(g_s[...], axis=1, keepdims=True)
        lp_ref[...] = g - (M + jnp.log(S))
        cand = jnp.where(m_l == M, i_s[...], IMAX)
        mode_ref[...] = jnp.min(cand, axis=1, keepdims=True)


_tc_reduce = pl.pallas_call(
    _tc_body,
    grid=(NBLK,),
    in_specs=[
        pl.BlockSpec((B, 1), lambda k: (0, 0)),
        pl.BlockSpec((B, BV), lambda k: (0, k)),
    ],
    out_specs=[
        pl.BlockSpec((B, 1), lambda k: (0, 0)),
        pl.BlockSpec((B, 1), lambda k: (0, 0)),
    ],
    out_shape=(
        jax.ShapeDtypeStruct((B, 1), jnp.float32),
        jax.ShapeDtypeStruct((B, 1), jnp.int32),
    ),
    scratch_shapes=[
        pltpu.VMEM((B, LANES), jnp.float32),
        pltpu.VMEM((B, LANES), jnp.float32),
        pltpu.VMEM((B, LANES), jnp.int32),
        pltpu.VMEM((B, LANES), jnp.float32),
    ],
    compiler_params=pltpu.CompilerParams(
        dimension_semantics=("arbitrary",),
    ),
)


@functools.partial(
    pl.kernel,
    out_type=jax.ShapeDtypeStruct((B,), jnp.int32),
    mesh=_mesh,
    scratch_types=(
        pltpu.VMEM((B,), jnp.int32),
        pltpu.VMEM((B,), jnp.int32),
        pltpu.SemaphoreType.DMA,
    ),
)
def _sc_idgather(actions_hbm, aout_hbm, idxbuf, gbuf, semg):
    wid = lax.axis_index("c") * 16 + lax.axis_index("s")

    @pl.when(wid == 0)
    def _():
        for h in range(B // 16):
            idxbuf[pl.ds(h * 16, 16)] = lax.iota(jnp.int32, 16) + h * 16
        pltpu.async_copy(actions_hbm.at[idxbuf], gbuf, semg).wait()
        pltpu.sync_copy(gbuf, aout_hbm)


def kernel(logits, actions):
    a32 = actions.reshape(B).astype(jnp.int32)
    a_sc = _sc_idgather(a32)
    lp, mode = _tc_reduce(a_sc.reshape(B, 1), logits)
    return (lp, mode)


# speedup vs baseline: 1.0258x; 1.0258x over previous
import functools

import jax
import jax.numpy as jnp
from jax import lax
from jax.experimental import pallas as pl
from jax.experimental.pallas import tpu as pltpu
from jax.experimental.pallas import tpu_sc as plsc

B = 32
V = 1_000_000
L = 16
LANES = 128
BV = 65536
G = BV // LANES
NBLK = (V + BV - 1) // BV
NEG_HUGE = -3.4028235e38
IMAX = 2**31 - 1

_mesh = plsc.VectorSubcoreMesh(core_axis_name="c", subcore_axis_name="s")


def _tc_body(a_ref, x_ref, logz_ref, mode_ref, gtab_ref, m_s, s_s, i_s, g_s):
    k = pl.program_id(0)

    @pl.when(k == 0)
    def _():
        m_s[...] = jnp.full((B, LANES), NEG_HUGE, jnp.float32)
        s_s[...] = jnp.zeros((B, LANES), jnp.float32)
        i_s[...] = jnp.zeros((B, LANES), jnp.int32)
        g_s[...] = jnp.zeros((B, LANES), jnp.float32)

    x3 = x_ref[...].reshape(B, G, LANES)
    offs = (k * BV
            + lax.broadcasted_iota(jnp.int32, (B, G, LANES), 1) * LANES
            + lax.broadcasted_iota(jnp.int32, (B, G, LANES), 2))
    x3 = jnp.where(offs < V, x3, NEG_HUGE)

    bm = jnp.max(x3, axis=1)
    m_old = m_s[...]
    m_new = jnp.maximum(m_old, bm)
    bs = jnp.sum(jnp.exp(x3 - m_new[:, None, :]), axis=1)
    s_s[...] = s_s[...] * jnp.exp(m_old - m_new) + bs
    m_s[...] = m_new

    a3 = a_ref[...][:, :, None]
    g_s[...] += jnp.sum(jnp.where(offs == a3, x3, 0.0), axis=1)

    giota = lax.broadcasted_iota(jnp.int32, (B, G, LANES), 1)
    bg = jnp.min(jnp.where(x3 == bm[:, None, :], giota, IMAX), axis=1)
    lane = lax.broadcasted_iota(jnp.int32, (B, LANES), 1)
    elem = k * BV + bg * LANES + lane
    i_s[...] = jnp.where(bm > m_old, elem, i_s[...])

    @pl.when(k == NBLK - 1)
    def _():
        m_l = m_s[...]
        M = jnp.max(m_l, axis=1, keepdims=True)
        S = jnp.sum(s_s[...] * jnp.exp(m_l - M), axis=1, keepdims=True)
        logz_ref[...] = M + jnp.log(S)
        cand = jnp.where(m_l == M, i_s[...], IMAX)
        mode_ref[...] = jnp.min(cand, axis=1, keepdims=True)
        gtab_ref[...] = g_s[...]


_tc_reduce = pl.pallas_call(
    _tc_body,
    grid=(NBLK,),
    in_specs=[
        pl.BlockSpec((B, 1), lambda k: (0, 0)),
        pl.BlockSpec((B, BV), lambda k: (0, k)),
    ],
    out_specs=[
        pl.BlockSpec((B, 1), lambda k: (0, 0)),
        pl.BlockSpec((B, 1), lambda k: (0, 0)),
        pl.BlockSpec((B, LANES), lambda k: (0, 0)),
    ],
    out_shape=(
        jax.ShapeDtypeStruct((B, 1), jnp.float32),
        jax.ShapeDtypeStruct((B, 1), jnp.int32),
        jax.ShapeDtypeStruct((B, LANES), jnp.float32),
    ),
    scratch_shapes=[
        pltpu.VMEM((B, LANES), jnp.float32),
        pltpu.VMEM((B, LANES), jnp.float32),
        pltpu.VMEM((B, LANES), jnp.int32),
        pltpu.VMEM((B, LANES), jnp.float32),
    ],
    compiler_params=pltpu.CompilerParams(
        dimension_semantics=("arbitrary",),
    ),
)


@functools.partial(
    pl.kernel,
    out_type=jax.ShapeDtypeStruct((B,), jnp.float32),
    mesh=_mesh,
    scratch_types=(
        pltpu.VMEM((B,), jnp.int32),
        pltpu.VMEM((B,), jnp.int32),
        pltpu.VMEM((B,), jnp.float32),
        pltpu.VMEM((B,), jnp.float32),
        pltpu.VMEM((B,), jnp.float32),
        pltpu.SemaphoreType.DMA,
    ),
)
def _sc_lp(gtab_hbm, actions_hbm, logz_hbm, lp_hbm,
           abuf, idxbuf, gbuf, zbuf, lpbuf, semg):
    wid = lax.axis_index("c") * 16 + lax.axis_index("s")

    @pl.when(wid == 0)
    def _():
        pltpu.sync_copy(actions_hbm, abuf)
        pltpu.sync_copy(logz_hbm, zbuf)
        for h in range(B // L):
            av = abuf[pl.ds(h * L, L)]
            rowbase = (lax.iota(jnp.int32, L) + h * L) * LANES
            idxbuf[pl.ds(h * L, L)] = rowbase + lax.rem(av, LANES)
        pltpu.async_copy(gtab_hbm.at[idxbuf], gbuf, semg).wait()
        for h in range(B // L):
            sl = pl.ds(h * L, L)
            lpbuf[sl] = gbuf[sl] - zbuf[sl]
        pltpu.sync_copy(lpbuf, lp_hbm)


def kernel(logits, actions):
    a32 = actions.reshape(B).astype(jnp.int32)
    logz, mode, gtab = _tc_reduce(a32.reshape(B, 1), logits)
    lp = _sc_lp(gtab.reshape(B * LANES), a32, logz.reshape(B))
    return (lp.reshape(B, 1), mode)


# speedup vs baseline: 1.1998x; 1.1696x over previous
import functools

import jax
import jax.numpy as jnp
from jax import lax
from jax.experimental import pallas as pl
from jax.experimental.pallas import tpu as pltpu
from jax.experimental.pallas import tpu_sc as plsc

B = 32
V = 1_000_000
L = 16
LANES = 128
BV = 65536
G = BV // LANES
NBLK = (V + BV - 1) // BV
NEG_HUGE = -3.4028235e38
IMAX = 2**31 - 1

_mesh = plsc.VectorSubcoreMesh(core_axis_name="c", subcore_axis_name="s")


def _tc_body(a_ref, x_ref, logz_ref, mode_ref, gtab_ref, m_s, s_s, i_s, g_s):
    k = pl.program_id(0)

    @pl.when(k == 0)
    def _():
        m_s[...] = jnp.full((B, LANES), NEG_HUGE, jnp.float32)
        s_s[...] = jnp.zeros((B, LANES), jnp.float32)
        i_s[...] = jnp.zeros((B, LANES), jnp.int32)
        g_s[...] = jnp.zeros((B, LANES), jnp.float32)

    def process(x3):
        bm = jnp.max(x3, axis=1)
        m_old = m_s[...]
        m_new = jnp.maximum(m_old, bm)
        bs = jnp.sum(jnp.exp(x3 - m_new[:, None, :]), axis=1)
        s_s[...] = s_s[...] * jnp.exp(m_old - m_new) + bs
        m_s[...] = m_new

        ag = a_ref[...][:, :, None] // LANES - k * G
        mg = lax.broadcasted_iota(jnp.int32, (B, G, 1), 1) == ag
        g_s[...] += jnp.sum(jnp.where(mg, x3, 0.0), axis=1)

        giota = lax.broadcasted_iota(jnp.int32, (B, G, 1), 1)
        bg = jnp.min(jnp.where(x3 == bm[:, None, :], giota, IMAX), axis=1)
        lane = lax.broadcasted_iota(jnp.int32, (B, LANES), 1)
        elem = k * BV + bg * LANES + lane
        i_s[...] = jnp.where(bm > m_old, elem, i_s[...])

    @pl.when(k < NBLK - 1)
    def _():
        process(x_ref[...].reshape(B, G, LANES))

    @pl.when(k == NBLK - 1)
    def _():
        x3 = x_ref[...].reshape(B, G, LANES)
        offs = (k * BV
                + lax.broadcasted_iota(jnp.int32, (B, G, LANES), 1) * LANES
                + lax.broadcasted_iota(jnp.int32, (B, G, LANES), 2))
        process(jnp.where(offs < V, x3, NEG_HUGE))

        m_l = m_s[...]
        M = jnp.max(m_l, axis=1, keepdims=True)
        S = jnp.sum(s_s[...] * jnp.exp(m_l - M), axis=1, keepdims=True)
        logz_ref[...] = M + jnp.log(S)
        cand = jnp.where(m_l == M, i_s[...], IMAX)
        mode_ref[...] = jnp.min(cand, axis=1, keepdims=True)
        gtab_ref[...] = g_s[...]


_tc_reduce = pl.pallas_call(
    _tc_body,
    grid=(NBLK,),
    in_specs=[
        pl.BlockSpec((B, 1), lambda k: (0, 0)),
        pl.BlockSpec((B, BV), lambda k: (0, k)),
    ],
    out_specs=[
        pl.BlockSpec((B, 1), lambda k: (0, 0)),
        pl.BlockSpec((B, 1), lambda k: (0, 0)),
        pl.BlockSpec((B, LANES), lambda k: (0, 0)),
    ],
    out_shape=(
        jax.ShapeDtypeStruct((B, 1), jnp.float32),
        jax.ShapeDtypeStruct((B, 1), jnp.int32),
        jax.ShapeDtypeStruct((B, LANES), jnp.float32),
    ),
    scratch_shapes=[
        pltpu.VMEM((B, LANES), jnp.float32),
        pltpu.VMEM((B, LANES), jnp.float32),
        pltpu.VMEM((B, LANES), jnp.int32),
        pltpu.VMEM((B, LANES), jnp.float32),
    ],
    compiler_params=pltpu.CompilerParams(
        dimension_semantics=("arbitrary",),
    ),
)


@functools.partial(
    pl.kernel,
    out_type=jax.ShapeDtypeStruct((B,), jnp.float32),
    mesh=_mesh,
    scratch_types=(
        pltpu.VMEM((B,), jnp.int32),
        pltpu.VMEM((B,), jnp.int32),
        pltpu.VMEM((B,), jnp.float32),
        pltpu.VMEM((B,), jnp.float32),
        pltpu.VMEM((B,), jnp.float32),
        pltpu.SemaphoreType.DMA,
    ),
)
def _sc_lp(gtab_hbm, actions_hbm, logz_hbm, lp_hbm,
           abuf, idxbuf, gbuf, zbuf, lpbuf, semg):
    wid = lax.axis_index("c") * 16 + lax.axis_index("s")

    @pl.when(wid == 0)
    def _():
        pltpu.sync_copy(actions_hbm, abuf)
        pltpu.sync_copy(logz_hbm, zbuf)
        for h in range(B // L):
            av = abuf[pl.ds(h * L, L)]
            rowbase = (lax.iota(jnp.int32, L) + h * L) * LANES
            idxbuf[pl.ds(h * L, L)] = rowbase + lax.rem(av, LANES)
        pltpu.async_copy(gtab_hbm.at[idxbuf], gbuf, semg).wait()
        for h in range(B // L):
            sl = pl.ds(h * L, L)
            lpbuf[sl] = gbuf[sl] - zbuf[sl]
        pltpu.sync_copy(lpbuf, lp_hbm)


def kernel(logits, actions):
    a32 = actions.reshape(B).astype(jnp.int32)
    logz, mode, gtab = _tc_reduce(a32.reshape(B, 1), logits)
    lp = _sc_lp(gtab.reshape(B * LANES), a32, logz.reshape(B))
    return (lp.reshape(B, 1), mode)
